# baseline (device time: 9307 ns/iter reference)
import jax
import jax.numpy as jnp
from jax import lax
from jax.experimental import pallas as pl
from jax.experimental.pallas import tpu as pltpu

N_DEV = 4


def _ce(x, idx, j, asc, flip=None):
    down = jnp.roll(x, -j, axis=1)
    up = jnp.roll(x, j, axis=1)
    lower = (idx & j) == 0
    partner = jnp.where(lower, down, up)
    take_min = asc == lower
    if flip is not None:
        take_min = jnp.logical_xor(take_min, flip)
    return jnp.where(take_min, jnp.minimum(x, partner), jnp.maximum(x, partner))


def kernel(x):
    return _pallas_sort_t(x.T).T


def _pallas_sort_t(xt):
    n, m_per = xt.shape
    m_full = N_DEV * m_per

    def body(xt_ref, out_ref, chunk_ref, full_ref, send_sems, recv_sems):
        my_pos = lax.axis_index("i")

        barrier_sem = pltpu.get_barrier_semaphore()
        for d in (2, 1, 3):
            pl.semaphore_signal(
                barrier_sem, inc=1,
                device_id=((my_pos + d) % N_DEV,),
                device_id_type=pl.DeviceIdType.MESH,
            )

        xv = xt_ref[:, :].astype(jnp.bfloat16)
        idx_m = lax.broadcasted_iota(jnp.int32, (n, m_per), 1)
        flip = (my_pos % 2) == 1
        k = 2
        while k <= m_per:
            j = k // 2
            while j >= 1:
                xv = _ce(xv, idx_m, j, (idx_m & k) == 0, flip)
                j //= 2
            k *= 2
        chunk_ref[:, :] = xv
        full_ref[:, pl.ds(my_pos * m_per, m_per)] = xv

        pl.semaphore_wait(barrier_sem, N_DEV - 1)

        sends = []
        for d in (2, 1, 3):
            rdma = pltpu.make_async_remote_copy(
                src_ref=chunk_ref,
                dst_ref=full_ref.at[:, pl.ds(my_pos * m_per, m_per)],
                send_sem=send_sems.at[d - 1],
                recv_sem=recv_sems.at[d - 1],
                device_id=((my_pos + d) % N_DEV,),
                device_id_type=pl.DeviceIdType.MESH,
            )
            rdma.start()
            sends.append(rdma)

        for d in range(1, N_DEV):
            origin = (my_pos - d) % N_DEV
            recv = pltpu.make_async_remote_copy(
                src_ref=chunk_ref,
                dst_ref=full_ref.at[:, pl.ds(origin * m_per, m_per)],
                send_sem=send_sems.at[d - 1],
                recv_sem=recv_sems.at[d - 1],
                device_id=(my_pos,),
                device_id_type=pl.DeviceIdType.MESH,
            )
            recv.wait_recv()

        xf = full_ref[:, :]
        idx_f = lax.broadcasted_iota(jnp.int32, (n, m_full), 1)
        asc_f = (idx_f & 256) == 0
        j = 128
        while j >= 1:
            xf = _ce(xf, idx_f, j, asc_f)
            j //= 2

        m_half = 2 * m_per
        lo = xf[:, :m_half]
        hi = xf[:, m_half:]
        is_lo_half = my_pos < 2
        xh = jnp.where(is_lo_half, jnp.minimum(lo, hi), jnp.maximum(lo, hi))

        a = xh[:, :m_per]
        b = xh[:, m_per:]
        is_lo_q = (my_pos % 2) == 0
        xq = jnp.where(is_lo_q, jnp.minimum(a, b), jnp.maximum(a, b))

        j = 64
        while j >= 1:
            xq = _ce(xq, idx_m, j, True)
            j //= 2
        out_ref[:, :] = xq

        for rdma in sends:
            rdma.wait_send()

    return pl.pallas_call(
        body,
        out_shape=jax.ShapeDtypeStruct((n, m_per), jnp.bfloat16),
        in_specs=[pl.BlockSpec(memory_space=pltpu.VMEM)],
        out_specs=pl.BlockSpec(memory_space=pltpu.VMEM),
        scratch_shapes=[
            pltpu.VMEM((n, m_per), jnp.bfloat16),
            pltpu.VMEM((n, m_full), jnp.bfloat16),
            pltpu.SemaphoreType.DMA((N_DEV - 1,)),
            pltpu.SemaphoreType.DMA((N_DEV - 1,)),
        ],
        compiler_params=pltpu.CompilerParams(collective_id=0),
    )(xt)
